# baseline (speedup 1.0000x reference)
import jax
import jax.numpy as jnp
from jax import lax
from jax.experimental import pallas as pl
from jax.experimental.pallas import tpu as pltpu
from jax.experimental.pallas import tpu_sc as plsc

B, S, D = 4, 8192, 1024
NC, NS = 2, 16
NW = NC * NS
POS_W = S // NW
R = 16
NBLK = POS_W // R
UNROLL = 4
TC_B = 3
TC_ROWS = 256


def _sc_body(x_hbm, w_hbm, out_hbm, *scr):
    wb = [scr[0], scr[2]]
    xb = [scr[1], scr[3]]
    in_sem = [scr[4], scr[5]]
    out_sem = [scr[6], scr[7]]

    wid = lax.axis_index("s") * NC + lax.axis_index("c")
    base = wid * POS_W

    def start_in(s, j):
        r0 = base + j * R
        pltpu.async_copy(w_hbm.at[pl.ds(r0, R)], wb[s], in_sem[s])
        pltpu.async_copy(x_hbm.at[pl.ds(r0, R)], xb[s], in_sem[s])

    def start_out(s, j):
        r0 = base + j * R
        pltpu.async_copy(xb[s], out_hbm.at[pl.ds(r0, R)], out_sem[s])

    def wait_in(s):
        pltpu.make_async_copy(w_hbm.at[pl.ds(0, R)], wb[s], in_sem[s]).wait()
        pltpu.make_async_copy(x_hbm.at[pl.ds(0, R)], xb[s], in_sem[s]).wait()

    def wait_out(s):
        pltpu.make_async_copy(xb[s], out_hbm.at[pl.ds(0, R)], out_sem[s]).wait()

    def compute(s):
        x0 = xb[s]
        w = wb[s]

        def add_chunk(i, c2):
            o = i * (16 * UNROLL)
            for u in range(UNROLL):
                sl = pl.ds(o + u * 16, 16)
                for r in range(R):
                    x0[r, sl] = x0[r, sl] + w[r, sl]
            return c2

        lax.fori_loop(0, D // (16 * UNROLL), add_chunk, 0)

    def process(j, s, has_next, has_prev_out):
        if has_next:
            if has_prev_out:
                wait_out(1 - s)
            start_in(1 - s, j + 1)
        wait_in(s)
        compute(s)
        start_out(s, j)

    start_in(0, 0)
    process(0, 0, True, False)

    def middle(t, c):
        j = 1 + 2 * t
        process(j, 1, True, True)
        process(j + 1, 0, True, True)
        return c

    lax.fori_loop(0, (NBLK - 2) // 2, middle, 0)
    process(NBLK - 1, 1, False, True)
    wait_out(0)
    wait_out(1)


def _sc_add(x2, w2):
    mesh = plsc.VectorSubcoreMesh(core_axis_name="c", subcore_axis_name="s")
    f = pl.kernel(
        _sc_body,
        out_type=jax.ShapeDtypeStruct((S, D), jnp.float32),
        mesh=mesh,
        scratch_types=(
            [pltpu.VMEM((R, D), jnp.float32) for _ in range(4)]
            + [pltpu.SemaphoreType.DMA] * 4
        ),
        compiler_params=pltpu.CompilerParams(use_tc_tiling_on_sc=True),
    )
    return f(x2, w2)


def _tc_body(x_ref, w_ref, o_ref):
    o_ref[...] = x_ref[...] + w_ref[...]


def _tc_add(x3, w3):
    grid = (S // TC_ROWS,)
    return pl.pallas_call(
        _tc_body,
        grid=grid,
        in_specs=[
            pl.BlockSpec((TC_B, TC_ROWS, D), lambda i: (0, i, 0)),
            pl.BlockSpec((1, TC_ROWS, D), lambda i: (0, i, 0)),
        ],
        out_specs=pl.BlockSpec((TC_B, TC_ROWS, D), lambda i: (0, i, 0)),
        out_shape=jax.ShapeDtypeStruct((TC_B, S, D), jnp.float32),
    )(x3, w3)


@jax.jit
def _pe_add(x, w):
    out_tc = _tc_add(x[:TC_B], w)
    out_sc = _sc_add(x[TC_B], w[0])
    return jnp.concatenate([out_tc, out_sc[None]], axis=0)


def kernel(x, weight):
    return _pe_add(x, weight)


# speedup vs baseline: 1.4188x; 1.4188x over previous
import jax
import jax.numpy as jnp
from jax import lax
from jax.experimental import pallas as pl
from jax.experimental.pallas import tpu as pltpu
from jax.experimental.pallas import tpu_sc as plsc

B, S, D = 4, 8192, 1024
NC, NS = 2, 16
NW = NC * NS
POS_W = S // NW
R = 16
NBLK = POS_W // R
CH = 8
NCH = R // CH
UNROLL = 4
TC_B = 3
TC_ROWS = 512
SC_BASE = TC_B * S


def _sc_body(x_hbm, w_hbm, out_hbm, *scr):
    wb = [scr[0], scr[2]]
    xb = [scr[1], scr[3]]
    in_sem = [scr[4], scr[5]]
    out_sem = [scr[6], scr[7]]

    wid = lax.axis_index("s") * NC + lax.axis_index("c")
    base = wid * POS_W

    def start_in(s, j):
        r0 = base + j * R
        for c in range(NCH):
            pltpu.async_copy(w_hbm.at[pl.ds(r0 + c * CH, CH)],
                             wb[s].at[pl.ds(c * CH, CH)], in_sem[s])
            pltpu.async_copy(x_hbm.at[pl.ds(SC_BASE + r0 + c * CH, CH)],
                             xb[s].at[pl.ds(c * CH, CH)], in_sem[s])

    def start_out(s, j):
        r0 = base + j * R
        for c in range(NCH):
            pltpu.async_copy(xb[s].at[pl.ds(c * CH, CH)],
                             out_hbm.at[pl.ds(r0 + c * CH, CH)], out_sem[s])

    def wait_in(s):
        for c in range(NCH):
            pltpu.make_async_copy(w_hbm.at[pl.ds(0, CH)],
                                  wb[s].at[pl.ds(c * CH, CH)], in_sem[s]).wait()
            pltpu.make_async_copy(x_hbm.at[pl.ds(0, CH)],
                                  xb[s].at[pl.ds(c * CH, CH)], in_sem[s]).wait()

    def wait_out(s):
        for c in range(NCH):
            pltpu.make_async_copy(xb[s].at[pl.ds(c * CH, CH)],
                                  out_hbm.at[pl.ds(0, CH)], out_sem[s]).wait()

    def compute(s):
        x0 = xb[s]
        w = wb[s]

        def add_chunk(i, c2):
            o = i * (16 * UNROLL)
            for u in range(UNROLL):
                sl = pl.ds(o + u * 16, 16)
                for r in range(R):
                    x0[r, sl] = x0[r, sl] + w[r, sl]
            return c2

        lax.fori_loop(0, D // (16 * UNROLL), add_chunk, 0)

    def process(j, s, has_next, has_prev_out):
        if has_next:
            if has_prev_out:
                wait_out(1 - s)
            start_in(1 - s, j + 1)
        wait_in(s)
        compute(s)
        start_out(s, j)

    start_in(0, 0)
    process(0, 0, True, False)

    def middle(t, c):
        j = 1 + 2 * t
        process(j, 1, True, True)
        process(j + 1, 0, True, True)
        return c

    lax.fori_loop(0, (NBLK - 2) // 2, middle, 0)
    process(NBLK - 1, 1, False, True)
    wait_out(0)
    wait_out(1)


def _sc_add(x2, w2):
    mesh = plsc.VectorSubcoreMesh(core_axis_name="c", subcore_axis_name="s")
    f = pl.kernel(
        _sc_body,
        out_type=jax.ShapeDtypeStruct((S, D), jnp.float32),
        mesh=mesh,
        scratch_types=(
            [pltpu.VMEM((R, D), jnp.float32) for _ in range(4)]
            + [pltpu.SemaphoreType.DMA] * 4
        ),
        compiler_params=pltpu.CompilerParams(use_tc_tiling_on_sc=True),
    )
    return f(x2, w2)


def _tc_body(x_ref, w_ref, o_ref):
    o_ref[...] = x_ref[...] + w_ref[...]


def _tc_add(x, w):
    grid = (S // TC_ROWS,)
    return pl.pallas_call(
        _tc_body,
        grid=grid,
        in_specs=[
            pl.BlockSpec((TC_B, TC_ROWS, D), lambda i: (0, i, 0)),
            pl.BlockSpec((1, TC_ROWS, D), lambda i: (0, i, 0)),
        ],
        out_specs=pl.BlockSpec((TC_B, TC_ROWS, D), lambda i: (0, i, 0)),
        out_shape=jax.ShapeDtypeStruct((TC_B, S, D), jnp.float32),
    )(x, w)


@jax.jit
def _pe_add(x, w):
    out_tc = _tc_add(x, w)
    out_sc = _sc_add(x.reshape(B * S, D), w[0])
    return jnp.concatenate([out_tc, out_sc[None]], axis=0)


def kernel(x, weight):
    return _pe_add(x, weight)


# speedup vs baseline: 3.1102x; 2.1921x over previous
import jax
import jax.numpy as jnp
from jax.experimental import pallas as pl

B, S, D = 4, 8192, 1024
TCR = 256


def _tc_body(x_ref, w_ref, o_ref):
    o_ref[...] = x_ref[...] + w_ref[...]


@jax.jit
def _pe_add(x, w):
    grid = (S // TCR,)
    return pl.pallas_call(
        _tc_body,
        grid=grid,
        in_specs=[
            pl.BlockSpec((B, TCR, D), lambda i: (0, i, 0)),
            pl.BlockSpec((1, TCR, D), lambda i: (0, i, 0)),
        ],
        out_specs=pl.BlockSpec((B, TCR, D), lambda i: (0, i, 0)),
        out_shape=jax.ShapeDtypeStruct((B, S, D), jnp.float32),
    )(x, w)


def kernel(x, weight):
    return _pe_add(x, weight)
